# baseline (speedup 1.0000x reference)
import functools

import jax
import jax.numpy as jnp
from jax import lax
from jax.experimental import pallas as pl
from jax.experimental.pallas import tpu as pltpu
from jax.experimental.pallas import tpu_sc as plsc

D_MODEL = 2048
HIDDEN = 128
N_EXPERTS = 16
TOP_K = 2

TOKEN_TILE = 2048
N_CHUNKS = 2

SC_CORES = 2
SC_SUBCORES = 16
SC_WORKERS = SC_CORES * SC_SUBCORES
LANES = 16


def _mlp_body(x_ref, w1_ref, w2_ref, logits_ref, logits_t_ref):
    x = x_ref[...]
    h = jax.lax.dot_general(
        x, w1_ref[...], (((1,), (0,)), ((), ())),
        preferred_element_type=jnp.float32,
    )
    h = h * (1.0 / (1.0 + jnp.exp(-h)))
    logits_ref[...] = jax.lax.dot_general(
        h, w2_ref[...], (((1,), (0,)), ((), ())),
        preferred_element_type=jnp.float32,
    )
    logits_t_ref[...] = jax.lax.dot_general(
        w2_ref[...], h, (((0,), (1,)), ((), ())),
        preferred_element_type=jnp.float32,
    )


@functools.partial(jax.jit, static_argnames=("interpret",))
def _mlp(x, w1, w2, interpret=False):
    n_tok = x.shape[0]
    grid = (n_tok // TOKEN_TILE,)
    return pl.pallas_call(
        _mlp_body,
        grid=grid,
        in_specs=[
            pl.BlockSpec((TOKEN_TILE, D_MODEL), lambda i: (i, 0)),
            pl.BlockSpec((D_MODEL, HIDDEN), lambda i: (0, 0)),
            pl.BlockSpec((HIDDEN, N_EXPERTS), lambda i: (0, 0)),
        ],
        out_specs=[
            pl.BlockSpec((TOKEN_TILE, N_EXPERTS), lambda i: (i, 0)),
            pl.BlockSpec((N_EXPERTS, TOKEN_TILE), lambda i: (0, i)),
        ],
        out_shape=[
            jax.ShapeDtypeStruct((n_tok, N_EXPERTS), jnp.float32),
            jax.ShapeDtypeStruct((N_EXPERTS, n_tok), jnp.float32),
        ],
        interpret=interpret,
    )(x, w1, w2)


_SC_MESH = plsc.VectorSubcoreMesh(core_axis_name="c", subcore_axis_name="s")


@functools.lru_cache(maxsize=None)
def _make_topk_sc(n_tok):
    chunk = n_tok // SC_WORKERS

    @functools.partial(
        pl.kernel,
        mesh=_SC_MESH,
        out_type=[
            jax.ShapeDtypeStruct((TOP_K, n_tok), jnp.float32),
            jax.ShapeDtypeStruct((TOP_K, n_tok), jnp.int32),
        ],
        scratch_types=[
            pltpu.VMEM((N_EXPERTS, chunk), jnp.float32),
            pltpu.VMEM((chunk,), jnp.float32),
            pltpu.VMEM((chunk,), jnp.float32),
            pltpu.VMEM((chunk,), jnp.int32),
            pltpu.VMEM((chunk,), jnp.int32),
        ],
    )
    def _topk_sc(lt_hbm, w_hbm, idx_hbm, lchunk, wa, wb, ia, ib):
        wid = lax.axis_index("s") * SC_CORES + lax.axis_index("c")
        base = wid * chunk
        pltpu.sync_copy(lt_hbm.at[:, pl.ds(base, chunk)], lchunk)
        for g in range(chunk // LANES):
            les = [lchunk[e, pl.ds(g * LANES, LANES)] for e in range(N_EXPERTS)]
            m1 = les[0]
            i1 = jnp.zeros((LANES,), jnp.int32)
            for e in range(1, N_EXPERTS):
                gt = les[e] > m1
                m1 = jnp.where(gt, les[e], m1)
                i1 = jnp.where(gt, e, i1)
            m2 = jnp.full((LANES,), -jnp.inf, jnp.float32)
            i2 = jnp.zeros((LANES,), jnp.int32)
            for e in range(N_EXPERTS):
                cand = jnp.where(i1 == e, -jnp.inf, les[e])
                gt = cand > m2
                m2 = jnp.where(gt, cand, m2)
                i2 = jnp.where(gt, e, i2)
            ee = jnp.exp(m2 - m1)
            r = 1.0 / (1.0 + ee)
            sl = pl.ds(g * LANES, LANES)
            wa[sl] = r
            wb[sl] = ee * r
            ia[sl] = i1
            ib[sl] = i2
        pltpu.sync_copy(wa, w_hbm.at[0, pl.ds(base, chunk)])
        pltpu.sync_copy(wb, w_hbm.at[1, pl.ds(base, chunk)])
        pltpu.sync_copy(ia, idx_hbm.at[0, pl.ds(base, chunk)])
        pltpu.sync_copy(ib, idx_hbm.at[1, pl.ds(base, chunk)])

    return _topk_sc


def kernel(hidden_states, W1, W2):
    b, s, d = hidden_states.shape
    n_tok = b * s
    x = hidden_states.reshape(n_tok, d)
    cs = n_tok // N_CHUNKS
    topk_sc = _make_topk_sc(cs)
    logits_c, w_c, idx_c = [], [], []
    for c in range(N_CHUNKS):
        lg, lt = _mlp(x[c * cs:(c + 1) * cs], W1, W2)
        wt, it = topk_sc(lt)
        logits_c.append(lg)
        w_c.append(wt)
        idx_c.append(it)
    logits = jnp.concatenate(logits_c, axis=0)
    w_t = jnp.concatenate(w_c, axis=1)
    idx_t = jnp.concatenate(idx_c, axis=1)
    return (
        w_t.T.reshape(b, s, TOP_K),
        idx_t.T.reshape(b, s, TOP_K),
        logits.reshape(b, s, N_EXPERTS),
    )


# speedup vs baseline: 2.0689x; 2.0689x over previous
import functools

import jax
import jax.numpy as jnp
from jax import lax
from jax.experimental import pallas as pl
from jax.experimental.pallas import tpu as pltpu
from jax.experimental.pallas import tpu_sc as plsc

D_MODEL = 2048
HIDDEN = 128
N_EXPERTS = 16
TOP_K = 2

TOKEN_TILE = 2048
N_CHUNKS = 2

SC_CORES = 2
SC_SUBCORES = 16
SC_WORKERS = SC_CORES * SC_SUBCORES
LANES = 16


def _mlp_body(x_ref, w1_ref, w2_ref, logits_ref, logits_t_ref):
    x = x_ref[...]
    h = jax.lax.dot_general(
        x, w1_ref[...], (((1,), (0,)), ((), ())),
        preferred_element_type=jnp.float32,
    )
    h = h * (1.0 / (1.0 + jnp.exp(-h)))
    logits_ref[...] = jax.lax.dot_general(
        h, w2_ref[...], (((1,), (0,)), ((), ())),
        preferred_element_type=jnp.float32,
    )
    logits_t_ref[...] = jax.lax.dot_general(
        w2_ref[...], h, (((0,), (1,)), ((), ())),
        preferred_element_type=jnp.float32,
    )


@functools.partial(jax.jit, static_argnames=("interpret", "chunk_id", "n_chunks"))
def _mlp(x, w1, w2, chunk_id=0, n_chunks=1, interpret=False):
    n_tok = x.shape[0] // n_chunks
    grid = (n_tok // TOKEN_TILE,)
    base_blk = chunk_id * grid[0]
    return pl.pallas_call(
        _mlp_body,
        grid=grid,
        in_specs=[
            pl.BlockSpec((TOKEN_TILE, D_MODEL), lambda i: (base_blk + i, 0)),
            pl.BlockSpec((D_MODEL, HIDDEN), lambda i: (0, 0)),
            pl.BlockSpec((HIDDEN, N_EXPERTS), lambda i: (0, 0)),
        ],
        out_specs=[
            pl.BlockSpec((TOKEN_TILE, N_EXPERTS), lambda i: (i, 0)),
            pl.BlockSpec((N_EXPERTS, TOKEN_TILE), lambda i: (0, i)),
        ],
        out_shape=[
            jax.ShapeDtypeStruct((n_tok, N_EXPERTS), jnp.float32),
            jax.ShapeDtypeStruct((N_EXPERTS, n_tok), jnp.float32),
        ],
        interpret=interpret,
    )(x, w1, w2)


_SC_MESH = plsc.VectorSubcoreMesh(core_axis_name="c", subcore_axis_name="s")


@functools.lru_cache(maxsize=None)
def _make_topk_sc(n_tok):
    chunk = n_tok // SC_WORKERS

    @functools.partial(
        pl.kernel,
        mesh=_SC_MESH,
        out_type=[
            jax.ShapeDtypeStruct((TOP_K, n_tok), jnp.float32),
            jax.ShapeDtypeStruct((TOP_K, n_tok), jnp.int32),
        ],
        scratch_types=[
            pltpu.VMEM((N_EXPERTS, chunk), jnp.float32),
            pltpu.VMEM((chunk,), jnp.float32),
            pltpu.VMEM((chunk,), jnp.float32),
            pltpu.VMEM((chunk,), jnp.int32),
            pltpu.VMEM((chunk,), jnp.int32),
        ],
    )
    def _topk_sc(lt_hbm, w_hbm, idx_hbm, lchunk, wa, wb, ia, ib):
        wid = lax.axis_index("s") * SC_CORES + lax.axis_index("c")
        base = wid * chunk
        pltpu.sync_copy(lt_hbm.at[:, pl.ds(base, chunk)], lchunk)
        for g in range(chunk // LANES):
            les = [lchunk[e, pl.ds(g * LANES, LANES)] for e in range(N_EXPERTS)]
            m1 = les[0]
            i1 = jnp.zeros((LANES,), jnp.int32)
            for e in range(1, N_EXPERTS):
                gt = les[e] > m1
                m1 = jnp.where(gt, les[e], m1)
                i1 = jnp.where(gt, e, i1)
            m2 = jnp.full((LANES,), -jnp.inf, jnp.float32)
            i2 = jnp.zeros((LANES,), jnp.int32)
            for e in range(N_EXPERTS):
                cand = jnp.where(i1 == e, -jnp.inf, les[e])
                gt = cand > m2
                m2 = jnp.where(gt, cand, m2)
                i2 = jnp.where(gt, e, i2)
            ee = jnp.exp(m2 - m1)
            r = 1.0 / (1.0 + ee)
            sl = pl.ds(g * LANES, LANES)
            wa[sl] = r
            wb[sl] = ee * r
            ia[sl] = i1
            ib[sl] = i2
        pltpu.sync_copy(wa, w_hbm.at[0, pl.ds(base, chunk)])
        pltpu.sync_copy(wb, w_hbm.at[1, pl.ds(base, chunk)])
        pltpu.sync_copy(ia, idx_hbm.at[0, pl.ds(base, chunk)])
        pltpu.sync_copy(ib, idx_hbm.at[1, pl.ds(base, chunk)])

    return _topk_sc


def kernel(hidden_states, W1, W2):
    b, s, d = hidden_states.shape
    n_tok = b * s
    x = hidden_states.reshape(n_tok, d)
    cs = n_tok // N_CHUNKS
    topk_sc = _make_topk_sc(cs)
    logits_c, w_c, idx_c = [], [], []
    for c in range(N_CHUNKS):
        lg, lt = _mlp(x, W1, W2, chunk_id=c, n_chunks=N_CHUNKS)
        wt, it = topk_sc(lt)
        logits_c.append(lg)
        w_c.append(wt)
        idx_c.append(it)
    logits = jnp.concatenate(logits_c, axis=0)
    w_t = jnp.concatenate(w_c, axis=1)
    idx_t = jnp.concatenate(idx_c, axis=1)
    return (
        w_t.T.reshape(b, s, TOP_K),
        idx_t.T.reshape(b, s, TOP_K),
        logits.reshape(b, s, N_EXPERTS),
    )


# speedup vs baseline: 2.9628x; 1.4321x over previous
import functools

import jax
import jax.numpy as jnp
from jax.experimental import pallas as pl

D_MODEL = 2048
HIDDEN = 128
N_EXPERTS = 16
TOP_K = 2

TOKEN_TILE = 2048


def _router_body(x_ref, w1_ref, w2_ref, logits_ref, w_ref, idx_ref):
    x = x_ref[...]
    h = jax.lax.dot_general(
        x, w1_ref[...], (((1,), (0,)), ((), ())),
        preferred_element_type=jnp.float32,
    )
    h = h * (1.0 / (1.0 + jnp.exp(-h)))
    logits = jax.lax.dot_general(
        h, w2_ref[...], (((1,), (0,)), ((), ())),
        preferred_element_type=jnp.float32,
    )
    logits_ref[...] = logits
    logits_t = jax.lax.dot_general(
        w2_ref[...], h, (((0,), (1,)), ((), ())),
        preferred_element_type=jnp.float32,
    )

    t = logits_t.shape[1]
    iota = jax.lax.broadcasted_iota(jnp.int32, (N_EXPERTS, t), 0)
    m1 = jnp.max(logits_t, axis=0, keepdims=True)
    i1 = jnp.min(jnp.where(logits_t == m1, iota, N_EXPERTS), axis=0, keepdims=True)
    masked = jnp.where(iota == i1, -jnp.inf, logits_t)
    m2 = jnp.max(masked, axis=0, keepdims=True)
    i2 = jnp.min(jnp.where(masked == m2, iota, N_EXPERTS), axis=0, keepdims=True)

    e = jnp.exp(m2 - m1)
    denom = 1.0 + e
    w_ref[...] = jnp.concatenate([1.0 / denom, e / denom], axis=0)
    idx_ref[...] = jnp.concatenate([i1, i2], axis=0)


@functools.partial(jax.jit, static_argnames=("interpret",))
def _router(x, w1, w2, interpret=False):
    n_tok = x.shape[0]
    grid = (n_tok // TOKEN_TILE,)
    return pl.pallas_call(
        _router_body,
        grid=grid,
        in_specs=[
            pl.BlockSpec((TOKEN_TILE, D_MODEL), lambda i: (i, 0)),
            pl.BlockSpec((D_MODEL, HIDDEN), lambda i: (0, 0)),
            pl.BlockSpec((HIDDEN, N_EXPERTS), lambda i: (0, 0)),
        ],
        out_specs=[
            pl.BlockSpec((TOKEN_TILE, N_EXPERTS), lambda i: (i, 0)),
            pl.BlockSpec((TOP_K, TOKEN_TILE), lambda i: (0, i)),
            pl.BlockSpec((TOP_K, TOKEN_TILE), lambda i: (0, i)),
        ],
        out_shape=[
            jax.ShapeDtypeStruct((n_tok, N_EXPERTS), jnp.float32),
            jax.ShapeDtypeStruct((TOP_K, n_tok), jnp.float32),
            jax.ShapeDtypeStruct((TOP_K, n_tok), jnp.int32),
        ],
        interpret=interpret,
    )(x, w1, w2)


def kernel(hidden_states, W1, W2):
    b, s, d = hidden_states.shape
    x = hidden_states.reshape(b * s, d)
    logits, w_t, idx_t = _router(x, W1, W2)
    return (
        w_t.T.reshape(b, s, TOP_K),
        idx_t.T.reshape(b, s, TOP_K),
        logits.reshape(b, s, N_EXPERTS),
    )
